# baseline (speedup 1.0000x reference)
import functools

import jax
import jax.numpy as jnp
from jax import lax
from jax.experimental import pallas as pl
from jax.experimental.pallas import tpu as pltpu
from jax.experimental.pallas import tpu_sc as plsc

_D = 64
_LANES = 16


def _sc_weights_body(idx_hbm, freq_hbm, out_hbm, idx_v, f_v, w_v, sem, *,
                     l_pad):
    cid = lax.axis_index("c")
    sid = lax.axis_index("s")

    @pl.when(jnp.logical_and(cid == 0, sid == 0))
    def _():
        pltpu.sync_copy(idx_hbm, idx_v)
        pltpu.async_copy(freq_hbm.at[idx_v], f_v, sem).wait()
        for k in range(l_pad // _LANES):
            sl = pl.ds(k * _LANES, _LANES)
            w_v[sl] = 1.0 / f_v[sl]
        pltpu.sync_copy(w_v, out_hbm)


def _make_sc_weights(l_pad):
    return functools.partial(
        pl.kernel,
        out_type=jax.ShapeDtypeStruct((l_pad,), jnp.float32),
        mesh=plsc.VectorSubcoreMesh(core_axis_name="c", subcore_axis_name="s"),
        scratch_types=[
            pltpu.VMEM((l_pad,), jnp.int32),
            pltpu.VMEM((l_pad,), jnp.float32),
            pltpu.VMEM((l_pad,), jnp.float32),
            pltpu.SemaphoreType.DMA,
        ],
        compiler_params=pltpu.CompilerParams(use_tc_tiling_on_sc=True),
    )(functools.partial(_sc_weights_body, l_pad=l_pad))


def _tc_body(idx_sref, w_sref, wt_ref, b_ref, emb_blk, out_ref, acc_ref, *,
             l_pad, scale):
    i = pl.program_id(0)

    @pl.when(i == 0)
    def _():
        acc_ref[...] = jnp.zeros_like(acc_ref)

    r = jax.lax.bitwise_and(idx_sref[i], 7)
    w = w_sref[i]
    acc_ref[...] += w * emb_blk[pl.ds(r, 1), :]

    @pl.when(i == l_pad - 1)
    def _():
        bow = acc_ref[...] * scale
        logits = lax.dot_general(
            bow, wt_ref[...], (((1,), (1,)), ((), ())),
            preferred_element_type=jnp.float32) + b_ref[...]
        m = jnp.max(logits, axis=-1, keepdims=True)
        s = logits - m
        lse = jnp.log(jnp.sum(jnp.exp(s), axis=-1, keepdims=True))
        out_ref[...] = s - lse


def kernel(input, emb_tensor, freq, W, b):
    L = input.shape[1]
    l_pad = ((L + _LANES - 1) // _LANES) * _LANES
    idx = jnp.concatenate(
        [input[0], jnp.zeros((l_pad - L,), jnp.int32)])
    w = _make_sc_weights(l_pad)(idx, freq)

    scale = 1.0 / (float(L) * 100000.0)
    grid_spec = pltpu.PrefetchScalarGridSpec(
        num_scalar_prefetch=1,
        grid=(l_pad,),
        in_specs=[
            pl.BlockSpec(memory_space=pltpu.SMEM),
            pl.BlockSpec((2, _D), lambda i, idx_ref: (0, 0)),
            pl.BlockSpec((1, 2), lambda i, idx_ref: (0, 0)),
            pl.BlockSpec((8, _D),
                         lambda i, idx_ref: (idx_ref[i] // 8, 0)),
        ],
        out_specs=pl.BlockSpec((1, 2), lambda i, idx_ref: (0, 0)),
        scratch_shapes=[pltpu.VMEM((1, _D), jnp.float32)],
    )
    out = pl.pallas_call(
        functools.partial(_tc_body, l_pad=l_pad, scale=scale),
        grid_spec=grid_spec,
        out_shape=jax.ShapeDtypeStruct((1, 2), jnp.float32),
    )(idx, w, W, b.reshape(1, 2), emb_tensor)
    return out
